# Initial kernel scaffold; baseline (speedup 1.0000x reference)
import jax
import jax.numpy as jnp
from jax.experimental import pallas as pl


def kernel(thin_edges):
    raise NotImplementedError("write your pallas kernel here")


# speedup vs baseline: 139.9400x; 139.9400x over previous
import jax
import jax.numpy as jnp
from jax.experimental import pallas as pl
from jax.experimental.pallas import tpu as pltpu

_LOW_T = 1.0
_HIGH_T = 3.0
_H = 224
_W = 224
_NLEV = 8


def _shift_lanes(a, s):
    if s == 0:
        return a
    z = jnp.zeros(a.shape[:-1] + (abs(s),), a.dtype)
    if s > 0:
        return jnp.concatenate([z, a[..., :-s]], axis=-1)
    return jnp.concatenate([a[..., -s:], z], axis=-1)


def _shift_rows(a, s):
    if s == 0:
        return a
    z = jnp.zeros((abs(s),) + a.shape[1:], a.dtype)
    if s > 0:
        return jnp.concatenate([z, a[:-s, :]], axis=0)
    return jnp.concatenate([a[-s:, :], z], axis=0)


def _hyst_body(x_ref, low_ref, high_ref, final_ref,
               w_ref, pre_ref, b_ref, pdn_ref, pup_ref):
    x = x_ref[...]
    low_ref[...] = jnp.where(x < _LOW_T, 0.0, x)
    high_ref[...] = jnp.where(x < _HIGH_T, 0.0, x)

    xt = x.T
    w = (xt >= _LOW_T).astype(jnp.int32)
    w_ref[...] = w
    pre_ref[...] = (xt >= _HIGH_T).astype(jnp.int32)

    p = w
    for k in range(_NLEV):
        pdn_ref[k] = p
        p = p & _shift_lanes(p, 1 << k)
    p = w
    for k in range(_NLEV):
        pup_ref[k] = p
        p = p & _shift_lanes(p, -(1 << k))

    iota_y = jax.lax.broadcasted_iota(jnp.int32, (1, _H), 1)
    ymask = ((iota_y >= 1) & (iota_y <= _H - 2)).astype(jnp.int32)

    def sweep(dx, dy):
        b_ref[...] = jnp.zeros((_W, _H), jnp.int32)
        pk_ref = pdn_ref if dy > 0 else pup_ref

        def body(i, b_prev):
            cx = (1 + i) if dx > 0 else (_W - 2 - i)
            wc = w_ref[pl.ds(cx, 1), :]
            prec = pre_ref[pl.ds(cx, 1), :]
            d = b_prev | _shift_lanes(b_prev, 1) | _shift_lanes(b_prev, -1)
            g = (prec | (wc & d)) & ymask
            for k in range(_NLEV):
                pk = pk_ref[k, pl.ds(cx, 1), :]
                g = g | (pk & _shift_lanes(g, dy * (1 << k)))
            g = g & ymask
            b_ref[pl.ds(cx, 1), :] = g
            return g

        jax.lax.fori_loop(0, _W - 2, body, jnp.zeros((1, _H), jnp.int32),
                          unroll=False)

        b = b_ref[...]
        dily = b | _shift_lanes(b, 1) | _shift_lanes(b, -1)
        dil = dily | _shift_rows(dily, 1) | _shift_rows(dily, -1)
        pre_ref[...] = pre_ref[...] | (w_ref[...] & dil)

    sweep(+1, +1)
    sweep(-1, -1)
    sweep(+1, -1)
    sweep(-1, +1)

    on = pre_ref[...].T
    final_ref[...] = jnp.where(on > 0, x, 0.0)


def _build_call(interpret=False):
    return pl.pallas_call(
        _hyst_body,
        out_shape=[
            jax.ShapeDtypeStruct((_H, _W), jnp.float32),
            jax.ShapeDtypeStruct((_H, _W), jnp.float32),
            jax.ShapeDtypeStruct((_H, _W), jnp.float32),
        ],
        scratch_shapes=[
            pltpu.VMEM((_W, _H), jnp.int32),
            pltpu.VMEM((_W, _H), jnp.int32),
            pltpu.VMEM((_W, _H), jnp.int32),
            pltpu.VMEM((_NLEV, _W, _H), jnp.int32),
            pltpu.VMEM((_NLEV, _W, _H), jnp.int32),
        ],
        interpret=interpret,
    )


@jax.jit
def _run(x2d):
    return _build_call()(x2d)


def kernel(thin_edges):
    x2d = thin_edges.reshape(_H, _W)
    low, high, final = _run(x2d)
    s = thin_edges.shape
    return low.reshape(s), high.reshape(s), final.reshape(s)


# speedup vs baseline: 4286.7694x; 30.6329x over previous
import jax
import jax.numpy as jnp
from jax.experimental import pallas as pl

_LOW_T = 1.0
_HIGH_T = 3.0
_H = 224
_W = 224
_NLEV = 8


def _shift_rows(a, s):
    if s == 0:
        return a
    z = jnp.zeros((abs(s),) + a.shape[1:], a.dtype)
    if s > 0:
        return jnp.concatenate([z, a[:-s, :]], axis=0)
    return jnp.concatenate([a[-s:, :], z], axis=0)


def _shift_lanes(a, s):
    if s == 0:
        return a
    z = jnp.zeros(a.shape[:-1] + (abs(s),), a.dtype)
    if s > 0:
        return jnp.concatenate([z, a[..., :-s]], axis=-1)
    return jnp.concatenate([a[..., -s:], z], axis=-1)


def _hyst_body(x_ref, low_ref, high_ref, final_ref):
    x = x_ref[...]
    low_ref[...] = jnp.where(x < _LOW_T, 0.0, x)
    high_ref[...] = jnp.where(x < _HIGH_T, 0.0, x)

    w = (x >= _LOW_T).astype(jnp.int32)
    pre = (x >= _HIGH_T).astype(jnp.int32)

    iy = jax.lax.broadcasted_iota(jnp.int32, (_H, _W), 0)
    ix = jax.lax.broadcasted_iota(jnp.int32, (_H, _W), 1)
    interior = ((iy >= 1) & (iy <= _H - 2) &
                (ix >= 1) & (ix <= _W - 2)).astype(jnp.int32)
    wm = w & interior

    pdn, pup = [], []
    p = wm
    for k in range(_NLEV):
        pdn.append(p)
        p = p & _shift_rows(p, 1 << k)
    p = wm
    for k in range(_NLEV):
        pup.append(p)
        p = p & _shift_rows(p, -(1 << k))

    for dx, dy in ((1, 1), (-1, -1), (1, -1), (-1, 1)):
        pk = pdn if dy > 0 else pup

        def yclose(b):
            for k in range(_NLEV):
                b = b | (pk[k] & _shift_rows(b, dy * (1 << k)))
            return b

        def step(b):
            d = b | _shift_rows(b, 1) | _shift_rows(b, -1)
            return yclose(b | (wm & _shift_lanes(d, dx)))

        def cond(c):
            return c[1]

        def body(c):
            b, _ = c
            prev = b
            for _ in range(4):
                b = step(b)
            return (b, jnp.any(b != prev))

        b0 = yclose(pre & interior)
        b, _ = jax.lax.while_loop(cond, body, (b0, jnp.bool_(True)))

        dily = b | _shift_rows(b, 1) | _shift_rows(b, -1)
        dil = dily | _shift_lanes(dily, 1) | _shift_lanes(dily, -1)
        pre = pre | (w & dil)

    final_ref[...] = jnp.where(pre > 0, x, 0.0)


def _build_call(interpret=False):
    return pl.pallas_call(
        _hyst_body,
        out_shape=[
            jax.ShapeDtypeStruct((_H, _W), jnp.float32),
            jax.ShapeDtypeStruct((_H, _W), jnp.float32),
            jax.ShapeDtypeStruct((_H, _W), jnp.float32),
        ],
        interpret=interpret,
    )


@jax.jit
def _run(x2d):
    return _build_call()(x2d)


def kernel(thin_edges):
    x2d = thin_edges.reshape(_H, _W)
    low, high, final = _run(x2d)
    s = thin_edges.shape
    return low.reshape(s), high.reshape(s), final.reshape(s)


# speedup vs baseline: 4968.4383x; 1.1590x over previous
import jax
import jax.numpy as jnp
from jax.experimental import pallas as pl

_LOW_T = 1.0
_HIGH_T = 3.0
_H = 224
_W = 224
_NLEV = 8


def _shift_rows(a, s):
    if s == 0:
        return a
    z = jnp.zeros((abs(s),) + a.shape[1:], a.dtype)
    if s > 0:
        return jnp.concatenate([z, a[:-s, :]], axis=0)
    return jnp.concatenate([a[-s:, :], z], axis=0)


def _shift_lanes(a, s):
    if s == 0:
        return a
    z = jnp.zeros(a.shape[:-1] + (abs(s),), a.dtype)
    if s > 0:
        return jnp.concatenate([z, a[..., :-s]], axis=-1)
    return jnp.concatenate([a[..., -s:], z], axis=-1)


def _hyst_body(x_ref, low_ref, high_ref, final_ref):
    x = x_ref[...]
    low_ref[...] = jnp.where(x < _LOW_T, 0.0, x)
    high_ref[...] = jnp.where(x < _HIGH_T, 0.0, x)

    w = (x >= _LOW_T).astype(jnp.int32)
    pre = (x >= _HIGH_T).astype(jnp.int32)

    iy = jax.lax.broadcasted_iota(jnp.int32, (_H, _W), 0)
    ix = jax.lax.broadcasted_iota(jnp.int32, (_H, _W), 1)
    interior = ((iy >= 1) & (iy <= _H - 2) &
                (ix >= 1) & (ix <= _W - 2)).astype(jnp.int32)
    wm = w & interior

    pdn, pup = [], []
    p = wm
    for k in range(_NLEV):
        pdn.append(p)
        p = p & _shift_rows(p, 1 << k)
    p = wm
    for k in range(_NLEV):
        pup.append(p)
        p = p & _shift_rows(p, -(1 << k))

    for dx, dy in ((1, 1), (-1, -1), (1, -1), (-1, 1)):
        pk = pdn if dy > 0 else pup

        def yclose(b, nlev):
            for k in range(nlev):
                b = b | (pk[k] & _shift_rows(b, dy * (1 << k)))
            return b

        def step(b):
            d = b | _shift_rows(b, 1) | _shift_rows(b, -1)
            return yclose(b | (wm & _shift_lanes(d, dx)), 4)

        def cond(c):
            return c[1]

        def body(c):
            b, _ = c
            prev = b
            for _ in range(4):
                b = step(b)
            return (b, jnp.any(b != prev))

        b0 = yclose(pre & interior, _NLEV)
        b, _ = jax.lax.while_loop(cond, body, (b0, jnp.bool_(True)))

        dily = b | _shift_rows(b, 1) | _shift_rows(b, -1)
        dil = dily | _shift_lanes(dily, 1) | _shift_lanes(dily, -1)
        pre = pre | (w & dil)

    final_ref[...] = jnp.where(pre > 0, x, 0.0)


def _build_call(interpret=False):
    return pl.pallas_call(
        _hyst_body,
        out_shape=[
            jax.ShapeDtypeStruct((_H, _W), jnp.float32),
            jax.ShapeDtypeStruct((_H, _W), jnp.float32),
            jax.ShapeDtypeStruct((_H, _W), jnp.float32),
        ],
        interpret=interpret,
    )


@jax.jit
def _run(x2d):
    return _build_call()(x2d)


def kernel(thin_edges):
    x2d = thin_edges.reshape(_H, _W)
    low, high, final = _run(x2d)
    s = thin_edges.shape
    return low.reshape(s), high.reshape(s), final.reshape(s)


# speedup vs baseline: 7997.9387x; 1.6097x over previous
import jax
import jax.numpy as jnp
from jax.experimental import pallas as pl

_LOW_T = 1.0
_HIGH_T = 3.0
_H = 224
_W = 224
_NW = 14
_BITS = 16
_MASK = (1 << _BITS) - 1
_NLEV = 8


def _shift_words(a, s):
    if s == 0:
        return a
    z = jnp.zeros((abs(s),) + a.shape[1:], a.dtype)
    if s > 0:
        return jnp.concatenate([z, a[:-s, :]], axis=0)
    return jnp.concatenate([a[-s:, :], z], axis=0)


def _shift_lanes(a, s):
    if s == 0:
        return a
    z = jnp.zeros(a.shape[:-1] + (abs(s),), a.dtype)
    if s > 0:
        return jnp.concatenate([z, a[..., :-s]], axis=-1)
    return jnp.concatenate([a[..., -s:], z], axis=-1)


def _shift_y(a, s):
    if s == 0:
        return a
    if s > 0:
        q, r = divmod(s, _BITS)
        if r == 0:
            return _shift_words(a, q)
        return (((_shift_words(a, q) << r) & _MASK)
                | (_shift_words(a, q + 1) >> (_BITS - r)))
    q, r = divmod(-s, _BITS)
    if r == 0:
        return _shift_words(a, -q)
    return ((_shift_words(a, -q) >> r)
            | ((_shift_words(a, -(q + 1)) << (_BITS - r)) & _MASK))


def _hyst_body(x_ref, low_ref, high_ref, final_ref):
    x = x_ref[...]
    low_ref[...] = jnp.where(x < _LOW_T, 0.0, x)
    high_ref[...] = jnp.where(x < _HIGH_T, 0.0, x)

    iy = jax.lax.broadcasted_iota(jnp.int32, (_H, _W), 0)
    ix = jax.lax.broadcasted_iota(jnp.int32, (_H, _W), 1)
    interior = ((iy >= 1) & (iy <= _H - 2) &
                (ix >= 1) & (ix <= _W - 2))

    wr = jax.lax.broadcasted_iota(jnp.int32, (_NW, _H), 0)
    yr = jax.lax.broadcasted_iota(jnp.int32, (_NW, _H), 1)
    sel = (yr // _BITS) == wr
    pk_mat = jnp.round(
        jnp.where(sel, jnp.exp2((yr - wr * _BITS).astype(jnp.float32)), 0.0)
    ).astype(jnp.bfloat16)

    def pack(mask_bool):
        m = mask_bool.astype(jnp.bfloat16)
        return jax.lax.dot_general(
            pk_mat, m, (((1,), (0,)), ((), ())),
            preferred_element_type=jnp.float32).astype(jnp.int32)

    wp = pack(x >= _LOW_T)
    wmp = pack((x >= _LOW_T) & interior)
    intp = pack(interior)
    pre = pack(x >= _HIGH_T)

    pdn, pup = [], []
    p = wmp
    for k in range(_NLEV):
        pdn.append(p)
        p = p & _shift_y(p, 1 << k)
    p = wmp
    for k in range(_NLEV):
        pup.append(p)
        p = p & _shift_y(p, -(1 << k))

    for dx, dy in ((1, 1), (-1, -1), (1, -1), (-1, 1)):
        pk = pdn if dy > 0 else pup

        def yclose(b, nlev):
            for k in range(nlev):
                b = b | (pk[k] & _shift_y(b, dy * (1 << k)))
            return b

        def step(b):
            d = b | _shift_y(b, 1) | _shift_y(b, -1)
            return yclose(b | (wmp & _shift_lanes(d, dx)), 4)

        def cond(c):
            return c[1]

        def body(c):
            b, _ = c
            prev = b
            for _ in range(4):
                b = step(b)
            return (b, jnp.any(b != prev))

        b0 = yclose(pre & intp, _NLEV)
        b, _ = jax.lax.while_loop(cond, body, (b0, jnp.bool_(True)))

        dily = b | _shift_y(b, 1) | _shift_y(b, -1)
        dil = dily | _shift_lanes(dily, 1) | _shift_lanes(dily, -1)
        pre = pre | (wp & dil)

    onp = pre
    rep = jnp.reshape(
        jnp.broadcast_to(onp[:, None, :], (_NW, _BITS, _W)), (_H, _W))
    bitsel = jnp.round(
        jnp.exp2((iy % _BITS).astype(jnp.float32))).astype(jnp.int32)
    on = (rep & bitsel) != 0
    final_ref[...] = jnp.where(on, x, 0.0)


def _build_call(interpret=False):
    return pl.pallas_call(
        _hyst_body,
        out_shape=[
            jax.ShapeDtypeStruct((_H, _W), jnp.float32),
            jax.ShapeDtypeStruct((_H, _W), jnp.float32),
            jax.ShapeDtypeStruct((_H, _W), jnp.float32),
        ],
        interpret=interpret,
    )


@jax.jit
def _run(x2d):
    return _build_call()(x2d)


def kernel(thin_edges):
    x2d = thin_edges.reshape(_H, _W)
    low, high, final = _run(x2d)
    s = thin_edges.shape
    return low.reshape(s), high.reshape(s), final.reshape(s)


# speedup vs baseline: 10374.9737x; 1.2972x over previous
import jax
import jax.numpy as jnp
from jax.experimental import pallas as pl

_LOW_T = 1.0
_HIGH_T = 3.0
_H = 224
_W = 224
_NW = 14
_BITS = 16
_MASK = (1 << _BITS) - 1
_NLEV = 8


def _shift_words(a, s):
    if s == 0:
        return a
    z = jnp.zeros((abs(s),) + a.shape[1:], a.dtype)
    if s > 0:
        return jnp.concatenate([z, a[:-s, :]], axis=0)
    return jnp.concatenate([a[-s:, :], z], axis=0)


def _shift_lanes(a, s):
    if s == 0:
        return a
    z = jnp.zeros(a.shape[:-1] + (abs(s),), a.dtype)
    if s > 0:
        return jnp.concatenate([z, a[..., :-s]], axis=-1)
    return jnp.concatenate([a[..., -s:], z], axis=-1)


def _shift_y(a, s):
    if s == 0:
        return a
    if s > 0:
        q, r = divmod(s, _BITS)
        if r == 0:
            return _shift_words(a, q)
        return (((_shift_words(a, q) << r) & _MASK)
                | (_shift_words(a, q + 1) >> (_BITS - r)))
    q, r = divmod(-s, _BITS)
    if r == 0:
        return _shift_words(a, -q)
    return ((_shift_words(a, -q) >> r)
            | ((_shift_words(a, -(q + 1)) << (_BITS - r)) & _MASK))


def _hyst_body(x_ref, low_ref, high_ref, final_ref):
    x = x_ref[...]
    low_ref[...] = jnp.where(x < _LOW_T, 0.0, x)
    high_ref[...] = jnp.where(x < _HIGH_T, 0.0, x)

    iy = jax.lax.broadcasted_iota(jnp.int32, (_H, _W), 0)
    ix = jax.lax.broadcasted_iota(jnp.int32, (_H, _W), 1)
    interior = ((iy >= 1) & (iy <= _H - 2) &
                (ix >= 1) & (ix <= _W - 2))

    wr = jax.lax.broadcasted_iota(jnp.int32, (_NW, _H), 0)
    yr = jax.lax.broadcasted_iota(jnp.int32, (_NW, _H), 1)
    sel = (yr // _BITS) == wr
    pk_mat = jnp.round(
        jnp.where(sel, jnp.exp2((yr - wr * _BITS).astype(jnp.float32)), 0.0)
    ).astype(jnp.bfloat16)

    def pack(mask_bool):
        m = mask_bool.astype(jnp.bfloat16)
        return jax.lax.dot_general(
            pk_mat, m, (((1,), (0,)), ((), ())),
            preferred_element_type=jnp.float32).astype(jnp.int32)

    wp = pack(x >= _LOW_T)
    wmp = pack((x >= _LOW_T) & interior)
    intp = pack(interior)
    pre = pack(x >= _HIGH_T)

    pdn, pup = [], []
    p = wmp
    for k in range(_NLEV):
        pdn.append(p)
        p = p & _shift_y(p, 1 << k)
    p = wmp
    for k in range(_NLEV):
        pup.append(p)
        p = p & _shift_y(p, -(1 << k))

    for dx, dy in ((1, 1), (-1, -1), (1, -1), (-1, 1)):
        pk = pdn if dy > 0 else pup

        def yclose(b, nlev):
            for k in range(nlev):
                b = b | (pk[k] & _shift_y(b, dy * (1 << k)))
            return b

        w1 = _shift_lanes(wmp, dx)
        c2 = {
            0: wmp & (w1 | _shift_y(w1, 1) | _shift_y(w1, -1)),
            1: wmp & (w1 | _shift_y(w1, -1)),
            -1: wmp & (w1 | _shift_y(w1, 1)),
            2: wmp & _shift_y(w1, -1),
            -2: wmp & _shift_y(w1, 1),
        }
        t4 = {}
        for o in range(-4, 5):
            acc = None
            for o2 in range(max(-2, o - 2), min(2, o + 2) + 1):
                term = c2[o2] & _shift_lanes(_shift_y(c2[o - o2], -o2), 2 * dx)
                acc = term if acc is None else (acc | term)
            t4[o] = acc
        wmp_s = _shift_lanes(wmp, -dx)
        c2s = {o: _shift_lanes(c2[o], -2 * dx) for o in c2}
        t4s = {o: _shift_lanes(t4[o], -4 * dx) for o in t4}

        def step(b):
            ss = {o: _shift_y(b, o) for o in range(-4, 5) if o != 0}
            ss[0] = b
            u1 = wmp_s & (b | ss[1] | ss[-1])
            u2 = None
            for o in range(-2, 3):
                term = c2s[o] & ss[-o]
                u2 = term if u2 is None else (u2 | term)
            u4 = None
            for o in range(-4, 5):
                term = t4s[o] & ss[-o]
                u4 = term if u4 is None else (u4 | term)
            adv = (_shift_lanes(u1, dx) | _shift_lanes(u2, 2 * dx)
                   | _shift_lanes(u4, 4 * dx))
            return yclose(b | adv, 4)

        def cond(c):
            return c[1]

        def body(c):
            b, _ = c
            prev = b
            for _ in range(2):
                b = step(b)
            return (b, jnp.any(b != prev))

        b0 = yclose(pre & intp, _NLEV)
        b, _ = jax.lax.while_loop(cond, body, (b0, jnp.bool_(True)))

        dily = b | _shift_y(b, 1) | _shift_y(b, -1)
        dil = dily | _shift_lanes(dily, 1) | _shift_lanes(dily, -1)
        pre = pre | (wp & dil)

    onp = pre
    rep = jnp.reshape(
        jnp.broadcast_to(onp[:, None, :], (_NW, _BITS, _W)), (_H, _W))
    bitsel = jnp.round(
        jnp.exp2((iy % _BITS).astype(jnp.float32))).astype(jnp.int32)
    on = (rep & bitsel) != 0
    final_ref[...] = jnp.where(on, x, 0.0)


def _build_call(interpret=False):
    return pl.pallas_call(
        _hyst_body,
        out_shape=[
            jax.ShapeDtypeStruct((_H, _W), jnp.float32),
            jax.ShapeDtypeStruct((_H, _W), jnp.float32),
            jax.ShapeDtypeStruct((_H, _W), jnp.float32),
        ],
        interpret=interpret,
    )


@jax.jit
def _run(x2d):
    return _build_call()(x2d)


def kernel(thin_edges):
    x2d = thin_edges.reshape(_H, _W)
    low, high, final = _run(x2d)
    s = thin_edges.shape
    return low.reshape(s), high.reshape(s), final.reshape(s)
